# baseline (speedup 1.0000x reference)
import functools

import jax
import jax.numpy as jnp
from jax import lax
from jax.experimental import pallas as pl
from jax.experimental.pallas import tpu as pltpu
from jax.experimental.pallas import tpu_sc as plsc

DM = 768
DFF = 3072
NE = 64
TOPK = 2
NT = 2048
NPAIR = NT * TOPK
TM = 256
TILES_MAX = (NPAIR + NE * TM) // TM
NMAX = TILES_MAX * TM
FF = 3072
NFF = DFF // FF

NW = 32
_SC_MESH = dict(core_axis_name="c", subcore_axis_name="s")


def _router_body(x_ref, wr_ref, br_ref, g0_ref, g1_ref, ps_ref,
                 te_ref, rb_ref, va_ref):
    x = x_ref[...]
    logits = jnp.dot(x, wr_ref[...], preferred_element_type=jnp.float32)
    logits = logits + br_ref[...]
    m = jnp.max(logits, axis=1, keepdims=True)
    ex = jnp.exp(logits - m)
    probs = ex / jnp.sum(ex, axis=1, keepdims=True)
    iota = lax.broadcasted_iota(jnp.int32, (NT, NE), 1)
    v1 = jnp.max(probs, axis=1, keepdims=True)
    i1 = jnp.min(jnp.where(probs == v1, iota, NE), axis=1, keepdims=True)
    masked = jnp.where(iota == i1, -1.0, probs)
    v2 = jnp.max(masked, axis=1, keepdims=True)
    i2 = jnp.min(jnp.where(masked == v2, iota, NE), axis=1, keepdims=True)
    s = v1 + v2
    g0_ref[...] = jnp.broadcast_to(v1 / s, (NT, 16))
    g1_ref[...] = jnp.broadcast_to(v2 / s, (NT, 16))

    e_cat = jnp.concatenate([i1, i2], axis=0)
    piota = lax.broadcasted_iota(jnp.int32, (NPAIR, NE), 1)
    oh = (e_cat == piota).astype(jnp.int32)
    c = oh
    k = 1
    while k < NPAIR:
        top = jnp.zeros((k, NE), jnp.int32)
        c = c + jnp.concatenate([top, c[:NPAIR - k]], axis=0)
        k *= 2
    rank = jnp.sum(oh * c, axis=1, keepdims=True) - 1
    counts = jnp.sum(oh, axis=0, keepdims=True)

    tiles_e = lax.shift_right_logical(counts + (TM - 1), TM.bit_length() - 1)
    cum = tiles_e
    k = 1
    while k < NE:
        left = jnp.zeros((1, k), jnp.int32)
        cum = cum + jnp.concatenate([left, cum[:, :NE - k]], axis=1)
        k *= 2
    tile_off = cum - tiles_e
    total = cum[:, NE - 1:]
    ps_ref[...] = TM * jnp.sum(oh * tile_off, axis=1, keepdims=True) + rank

    t_col = lax.broadcasted_iota(jnp.int32, (TILES_MAX, 1), 0)
    t_mat = lax.broadcasted_iota(jnp.int32, (TILES_MAX, NE), 0)
    expert_of_tile = jnp.sum((jnp.broadcast_to(cum, (TILES_MAX, NE)) <=
                              t_mat).astype(jnp.int32), axis=1, keepdims=True)
    eiota = lax.broadcasted_iota(jnp.int32, (1, NE), 1)
    e_last = jnp.max(jnp.where(counts > 0, eiota, -1), axis=1, keepdims=True)
    valid = t_col < total
    te_ref[...] = jnp.where(valid, jnp.clip(expert_of_tile, 0, NE - 1), e_last)
    rb_ref[...] = jnp.where(valid, t_col, total - 1)
    va_ref[...] = valid.astype(jnp.int32)


def _router(xf, Wr, br):
    return pl.pallas_call(
        _router_body,
        out_shape=(
            jax.ShapeDtypeStruct((NT, 16), jnp.float32),
            jax.ShapeDtypeStruct((NT, 16), jnp.float32),
            jax.ShapeDtypeStruct((NPAIR, 1), jnp.int32),
            jax.ShapeDtypeStruct((TILES_MAX, 1), jnp.int32),
            jax.ShapeDtypeStruct((TILES_MAX, 1), jnp.int32),
            jax.ShapeDtypeStruct((TILES_MAX, 1), jnp.int32),
        ),
    )(xf, Wr, br.reshape(1, NE))


_DISP_PER_W = NPAIR // NW


def _dispatch_body(xf_hbm, ps_hbm, out_hbm, idx_v, rows_v, sem):
    wid = lax.axis_index("s") * 2 + lax.axis_index("c")
    pbase = pl.multiple_of(wid * _DISP_PER_W, _DISP_PER_W)
    tbase = pl.multiple_of(jnp.remainder(wid, NW // 2) * _DISP_PER_W,
                           _DISP_PER_W)
    pltpu.sync_copy(ps_hbm.at[pl.ds(pbase, _DISP_PER_W)], idx_v)
    pltpu.sync_copy(xf_hbm.at[pl.ds(tbase, _DISP_PER_W)], rows_v)
    pltpu.async_copy(rows_v, out_hbm.at[idx_v], sem).wait()


def _dispatch(xf, ps):
    k = functools.partial(
        pl.kernel,
        mesh=plsc.VectorSubcoreMesh(**_SC_MESH),
        out_type=jax.ShapeDtypeStruct((NMAX, DM), jnp.float32),
        scratch_types=[
            pltpu.VMEM((_DISP_PER_W,), jnp.int32),
            pltpu.VMEM((_DISP_PER_W, DM), jnp.float32),
            pltpu.SemaphoreType.DMA,
        ],
    )(_dispatch_body)
    return k(xf, ps)


def _ffn_body(te_ref, rb_ref, va_ref, x_ref, w1_ref, b1_ref, w2_ref, b2_ref,
              y_ref):
    t = pl.program_id(0)
    f = pl.program_id(1)

    @pl.when(va_ref[t] == 1)
    def _():
        x = x_ref[...]
        h = jnp.dot(x, w1_ref[0], preferred_element_type=jnp.float32)
        h = h + b1_ref[0]
        h = 0.5 * h * (1.0 + lax.erf(h * 0.7071067811865476))
        yp = jnp.dot(h, w2_ref[0], preferred_element_type=jnp.float32)

        @pl.when(f == 0)
        def _():
            y_ref[...] = yp

        @pl.when(f != 0)
        def _():
            y_ref[...] = y_ref[...] + yp

        @pl.when(f == NFF - 1)
        def _():
            y_ref[...] = y_ref[...] + b2_ref[0]


def _frozen_f(va_ref, t, f):
    return jnp.where(va_ref[t] == 1, f, NFF - 1)


def _ffn(X_sorted, W1, b1, W2, b2, tile_expert, row_block, valid):
    grid_spec = pltpu.PrefetchScalarGridSpec(
        num_scalar_prefetch=3,
        grid=(TILES_MAX, NFF),
        in_specs=[
            pl.BlockSpec((TM, DM), lambda t, f, te, rb, va: (rb[t], 0)),
            pl.BlockSpec((1, DM, FF),
                         lambda t, f, te, rb, va: (te[t], 0, _frozen_f(va, t, f))),
            pl.BlockSpec((1, 1, FF),
                         lambda t, f, te, rb, va: (te[t], 0, _frozen_f(va, t, f))),
            pl.BlockSpec((1, FF, DM),
                         lambda t, f, te, rb, va: (te[t], _frozen_f(va, t, f), 0)),
            pl.BlockSpec((1, 1, DM), lambda t, f, te, rb, va: (te[t], 0, 0)),
        ],
        out_specs=pl.BlockSpec((TM, DM), lambda t, f, te, rb, va: (rb[t], 0)),
    )
    return pl.pallas_call(
        _ffn_body,
        grid_spec=grid_spec,
        out_shape=jax.ShapeDtypeStruct((NMAX, DM), jnp.float32),
    )(tile_expert, row_block, valid, X_sorted, W1, b1.reshape(NE, 1, DFF),
      W2, b2.reshape(NE, 1, DM))


_COMB_PER_W = NT // NW


def _combine_body(y_hbm, p0_hbm, p1_hbm, g0_hbm, g1_hbm, out_hbm,
                  i0_v, i1_v, g0_v, g1_v, r0_v, r1_v, sem):
    wid = lax.axis_index("s") * 2 + lax.axis_index("c")
    base = pl.multiple_of(wid * _COMB_PER_W, _COMB_PER_W)
    pltpu.sync_copy(p0_hbm.at[pl.ds(base, _COMB_PER_W)], i0_v)
    pltpu.sync_copy(p1_hbm.at[pl.ds(base, _COMB_PER_W)], i1_v)
    pltpu.sync_copy(g0_hbm.at[pl.ds(base, _COMB_PER_W)], g0_v)
    pltpu.sync_copy(g1_hbm.at[pl.ds(base, _COMB_PER_W)], g1_v)
    pltpu.async_copy(y_hbm.at[i0_v], r0_v, sem).wait()
    pltpu.async_copy(y_hbm.at[i1_v], r1_v, sem).wait()

    def row(rr, _):
        g0b = g0_v[rr, :]
        g1b = g1_v[rr, :]
        for cc in range(DM // 16):
            sl = pl.ds(cc * 16, 16)
            r0_v[rr, sl] = r0_v[rr, sl] * g0b + r1_v[rr, sl] * g1b
        return 0

    lax.fori_loop(0, _COMB_PER_W, row, 0)
    pltpu.sync_copy(r0_v, out_hbm.at[pl.ds(base, _COMB_PER_W)])


def _combine(Y, pos0, pos1, g0, g1):
    k = functools.partial(
        pl.kernel,
        mesh=plsc.VectorSubcoreMesh(**_SC_MESH),
        out_type=jax.ShapeDtypeStruct((NT, DM), jnp.float32),
        scratch_types=[
            pltpu.VMEM((_COMB_PER_W,), jnp.int32),
            pltpu.VMEM((_COMB_PER_W,), jnp.int32),
            pltpu.VMEM((_COMB_PER_W, 16), jnp.float32),
            pltpu.VMEM((_COMB_PER_W, 16), jnp.float32),
            pltpu.VMEM((_COMB_PER_W, DM), jnp.float32),
            pltpu.VMEM((_COMB_PER_W, DM), jnp.float32),
            pltpu.SemaphoreType.DMA,
        ],
    )(_combine_body)
    return k(Y, pos0, pos1, g0, g1)


def kernel(x, Wr, br, W1, b1, W2, b2):
    B, S, D = x.shape
    xf = x.reshape(-1, D)
    g0x, g1x, ps2, te2, rb2, va2 = _router(xf, Wr, br)
    ps = ps2.reshape(NPAIR)
    X_sorted = _dispatch(xf, ps)
    Y = _ffn(X_sorted, W1, b1, W2, b2, te2.reshape(TILES_MAX),
             rb2.reshape(TILES_MAX), va2.reshape(TILES_MAX))
    out = _combine(Y, ps[:NT], ps[NT:], g0x, g1x)
    return out.reshape(B, S, D)


# speedup vs baseline: 1.0316x; 1.0316x over previous
import functools

import jax
import jax.numpy as jnp
from jax import lax
from jax.experimental import pallas as pl
from jax.experimental.pallas import tpu as pltpu
from jax.experimental.pallas import tpu_sc as plsc

DM = 768
DFF = 3072
NE = 64
TOPK = 2
NT = 2048
NPAIR = NT * TOPK
TM = 128
TILES_MAX = (NPAIR + NE * TM) // TM
NMAX = TILES_MAX * TM
FF = 3072
NFF = DFF // FF

NW = 32
_SC_MESH = dict(core_axis_name="c", subcore_axis_name="s")


def _router_body(x_ref, wr_ref, br_ref, g0_ref, g1_ref, ps_ref,
                 te_ref, rb_ref, va_ref):
    x = x_ref[...]
    logits = jnp.dot(x, wr_ref[...], preferred_element_type=jnp.float32)
    logits = logits + br_ref[...]
    m = jnp.max(logits, axis=1, keepdims=True)
    ex = jnp.exp(logits - m)
    probs = ex / jnp.sum(ex, axis=1, keepdims=True)
    iota = lax.broadcasted_iota(jnp.int32, (NT, NE), 1)
    v1 = jnp.max(probs, axis=1, keepdims=True)
    i1 = jnp.min(jnp.where(probs == v1, iota, NE), axis=1, keepdims=True)
    masked = jnp.where(iota == i1, -1.0, probs)
    v2 = jnp.max(masked, axis=1, keepdims=True)
    i2 = jnp.min(jnp.where(masked == v2, iota, NE), axis=1, keepdims=True)
    s = v1 + v2
    g0_ref[...] = jnp.broadcast_to(v1 / s, (NT, 16))
    g1_ref[...] = jnp.broadcast_to(v2 / s, (NT, 16))

    e_cat = jnp.concatenate([i1, i2], axis=0)
    piota = lax.broadcasted_iota(jnp.int32, (NPAIR, NE), 1)
    oh = (e_cat == piota).astype(jnp.int32)
    c = oh
    k = 1
    while k < NPAIR:
        top = jnp.zeros((k, NE), jnp.int32)
        c = c + jnp.concatenate([top, c[:NPAIR - k]], axis=0)
        k *= 2
    rank = jnp.sum(oh * c, axis=1, keepdims=True) - 1
    counts = jnp.sum(oh, axis=0, keepdims=True)

    tiles_e = lax.shift_right_logical(counts + (TM - 1), TM.bit_length() - 1)
    cum = tiles_e
    k = 1
    while k < NE:
        left = jnp.zeros((1, k), jnp.int32)
        cum = cum + jnp.concatenate([left, cum[:, :NE - k]], axis=1)
        k *= 2
    tile_off = cum - tiles_e
    total = cum[:, NE - 1:]
    ps_ref[...] = TM * jnp.sum(oh * tile_off, axis=1, keepdims=True) + rank

    t_col = lax.broadcasted_iota(jnp.int32, (TILES_MAX, 1), 0)
    t_mat = lax.broadcasted_iota(jnp.int32, (TILES_MAX, NE), 0)
    expert_of_tile = jnp.sum((jnp.broadcast_to(cum, (TILES_MAX, NE)) <=
                              t_mat).astype(jnp.int32), axis=1, keepdims=True)
    eiota = lax.broadcasted_iota(jnp.int32, (1, NE), 1)
    e_last = jnp.max(jnp.where(counts > 0, eiota, -1), axis=1, keepdims=True)
    valid = t_col < total
    te_ref[...] = jnp.where(valid, jnp.clip(expert_of_tile, 0, NE - 1), e_last)
    rb_ref[...] = jnp.where(valid, t_col, total - 1)
    va_ref[...] = valid.astype(jnp.int32)


def _router(xf, Wr, br):
    return pl.pallas_call(
        _router_body,
        out_shape=(
            jax.ShapeDtypeStruct((NT, 16), jnp.float32),
            jax.ShapeDtypeStruct((NT, 16), jnp.float32),
            jax.ShapeDtypeStruct((NPAIR, 1), jnp.int32),
            jax.ShapeDtypeStruct((TILES_MAX, 1), jnp.int32),
            jax.ShapeDtypeStruct((TILES_MAX, 1), jnp.int32),
            jax.ShapeDtypeStruct((TILES_MAX, 1), jnp.int32),
        ),
    )(xf, Wr, br.reshape(1, NE))


_DISP_PER_W = NPAIR // NW


def _dispatch_body(xf_hbm, ps_hbm, out_hbm, idx_v, rows_v, sem, sem2):
    wid = lax.axis_index("s") * 2 + lax.axis_index("c")
    pbase = pl.multiple_of(wid * _DISP_PER_W, _DISP_PER_W)
    tbase = pl.multiple_of(jnp.remainder(wid, NW // 2) * _DISP_PER_W,
                           _DISP_PER_W)
    c1 = pltpu.async_copy(ps_hbm.at[pl.ds(pbase, _DISP_PER_W)], idx_v, sem)
    c2 = pltpu.async_copy(xf_hbm.at[pl.ds(tbase, _DISP_PER_W)], rows_v, sem2)
    c1.wait()
    c2.wait()
    pltpu.async_copy(rows_v, out_hbm.at[idx_v], sem).wait()


def _dispatch(xf, ps):
    k = functools.partial(
        pl.kernel,
        mesh=plsc.VectorSubcoreMesh(**_SC_MESH),
        out_type=jax.ShapeDtypeStruct((NMAX, DM), jnp.float32),
        scratch_types=[
            pltpu.VMEM((_DISP_PER_W,), jnp.int32),
            pltpu.VMEM((_DISP_PER_W, DM), jnp.float32),
            pltpu.SemaphoreType.DMA,
            pltpu.SemaphoreType.DMA,
        ],
    )(_dispatch_body)
    return k(xf, ps)


def _ffn_body(te_ref, rb_ref, va_ref, x_ref, w1_ref, b1_ref, w2_ref, b2_ref,
              y_ref):
    t = pl.program_id(0)
    f = pl.program_id(1)

    @pl.when(va_ref[t] == 1)
    def _():
        x = x_ref[...]
        h = jnp.dot(x, w1_ref[0], preferred_element_type=jnp.float32)
        h = h + b1_ref[0]
        h = 0.5 * h * (1.0 + lax.erf(h * 0.7071067811865476))
        yp = jnp.dot(h, w2_ref[0], preferred_element_type=jnp.float32)

        @pl.when(f == 0)
        def _():
            y_ref[...] = yp

        @pl.when(f != 0)
        def _():
            y_ref[...] = y_ref[...] + yp

        @pl.when(f == NFF - 1)
        def _():
            y_ref[...] = y_ref[...] + b2_ref[0]


def _frozen_f(va_ref, t, f):
    return jnp.where(va_ref[t] == 1, f, NFF - 1)


def _ffn(X_sorted, W1, b1, W2, b2, tile_expert, row_block, valid):
    grid_spec = pltpu.PrefetchScalarGridSpec(
        num_scalar_prefetch=3,
        grid=(TILES_MAX, NFF),
        in_specs=[
            pl.BlockSpec((TM, DM), lambda t, f, te, rb, va: (rb[t], 0)),
            pl.BlockSpec((1, DM, FF),
                         lambda t, f, te, rb, va: (te[t], 0, _frozen_f(va, t, f))),
            pl.BlockSpec((1, 1, FF),
                         lambda t, f, te, rb, va: (te[t], 0, _frozen_f(va, t, f))),
            pl.BlockSpec((1, FF, DM),
                         lambda t, f, te, rb, va: (te[t], _frozen_f(va, t, f), 0)),
            pl.BlockSpec((1, 1, DM), lambda t, f, te, rb, va: (te[t], 0, 0)),
        ],
        out_specs=pl.BlockSpec((TM, DM), lambda t, f, te, rb, va: (rb[t], 0)),
    )
    return pl.pallas_call(
        _ffn_body,
        grid_spec=grid_spec,
        out_shape=jax.ShapeDtypeStruct((NMAX, DM), jnp.float32),
    )(tile_expert, row_block, valid, X_sorted, W1, b1.reshape(NE, 1, DFF),
      W2, b2.reshape(NE, 1, DM))


_COMB_PER_W = NT // NW


def _combine_body(y_hbm, p0_hbm, p1_hbm, g0_hbm, g1_hbm, out_hbm,
                  i0_v, i1_v, g0_v, g1_v, r0_v, r1_v, sem, sem2):
    wid = lax.axis_index("s") * 2 + lax.axis_index("c")
    base = pl.multiple_of(wid * _COMB_PER_W, _COMB_PER_W)
    c0 = pltpu.async_copy(p0_hbm.at[pl.ds(base, _COMB_PER_W)], i0_v, sem)
    c1 = pltpu.async_copy(p1_hbm.at[pl.ds(base, _COMB_PER_W)], i1_v, sem2)
    c2 = pltpu.async_copy(g0_hbm.at[pl.ds(base, _COMB_PER_W)], g0_v, sem)
    c3 = pltpu.async_copy(g1_hbm.at[pl.ds(base, _COMB_PER_W)], g1_v, sem2)
    c0.wait()
    c1.wait()
    r0c = pltpu.async_copy(y_hbm.at[i0_v], r0_v, sem)
    r1c = pltpu.async_copy(y_hbm.at[i1_v], r1_v, sem2)
    c2.wait()
    c3.wait()
    r0c.wait()
    r1c.wait()

    def row(rr, _):
        g0b = g0_v[rr, :]
        g1b = g1_v[rr, :]
        for cc in range(DM // 16):
            sl = pl.ds(cc * 16, 16)
            r0_v[rr, sl] = r0_v[rr, sl] * g0b + r1_v[rr, sl] * g1b
        return 0

    lax.fori_loop(0, _COMB_PER_W, row, 0)
    pltpu.sync_copy(r0_v, out_hbm.at[pl.ds(base, _COMB_PER_W)])


def _combine(Y, pos0, pos1, g0, g1):
    k = functools.partial(
        pl.kernel,
        mesh=plsc.VectorSubcoreMesh(**_SC_MESH),
        out_type=jax.ShapeDtypeStruct((NT, DM), jnp.float32),
        scratch_types=[
            pltpu.VMEM((_COMB_PER_W,), jnp.int32),
            pltpu.VMEM((_COMB_PER_W,), jnp.int32),
            pltpu.VMEM((_COMB_PER_W, 16), jnp.float32),
            pltpu.VMEM((_COMB_PER_W, 16), jnp.float32),
            pltpu.VMEM((_COMB_PER_W, DM), jnp.float32),
            pltpu.VMEM((_COMB_PER_W, DM), jnp.float32),
            pltpu.SemaphoreType.DMA,
            pltpu.SemaphoreType.DMA,
        ],
    )(_combine_body)
    return k(Y, pos0, pos1, g0, g1)


def kernel(x, Wr, br, W1, b1, W2, b2):
    B, S, D = x.shape
    xf = x.reshape(-1, D)
    g0x, g1x, ps2, te2, rb2, va2 = _router(xf, Wr, br)
    ps = ps2.reshape(NPAIR)
    X_sorted = _dispatch(xf, ps)
    Y = _ffn(X_sorted, W1, b1, W2, b2, te2.reshape(TILES_MAX),
             rb2.reshape(TILES_MAX), va2.reshape(TILES_MAX))
    out = _combine(Y, ps[:NT], ps[NT:], g0x, g1x)
    return out.reshape(B, S, D)
